# baseline (speedup 1.0000x reference)
import functools
import numpy as np
import jax
import jax.numpy as jnp
from jax import lax
from jax.experimental import pallas as pl
from jax.experimental.pallas import tpu as pltpu
from jax.experimental.pallas import tpu_sc as plsc

NUM_NODES = 100000
EMBED_DIM = 128
BATCH = 10000
NUM_NEIGHBORS = 32

_NC, _NS = 2, 16
_NW = _NC * _NS
_CHUNK_IDX = 128
_B_PER_CHUNK = _CHUNK_IDX // NUM_NEIGHBORS
_N_CHUNKS = BATCH * NUM_NEIGHBORS // _CHUNK_IDX
_CHUNKS_LO = _N_CHUNKS // _NW
_N_HI = _N_CHUNKS - _CHUNKS_LO * _NW
_NBUF = 2

_COL_PERM = np.concatenate(
    [32 * g + np.arange(32).reshape(2, 16).T.reshape(-1) for g in range(4)])


def _tec_body(theta_hbm, idx_hbm, out_hbm, idx_v, rows0, rows1, oc0, oc1,
              gsem0, gsem1, ssem0, ssem1):
    wid = lax.axis_index("s") * _NC + lax.axis_index("c")
    is_hi = wid < _N_HI
    start_chunk = jnp.where(is_hi, (_CHUNKS_LO + 1) * wid,
                            (_CHUNKS_LO + 1) * _N_HI
                            + _CHUNKS_LO * (wid - _N_HI))
    n_chunks = jnp.where(is_hi, _CHUNKS_LO + 1, _CHUNKS_LO)

    pltpu.sync_copy(idx_hbm.at[pl.ds(start_chunk, _CHUNKS_LO)],
                    idx_v.at[pl.ds(0, _CHUNKS_LO)])

    @pl.when(is_hi)
    def _():
        pltpu.sync_copy(idx_hbm.at[pl.ds(start_chunk + _CHUNKS_LO, 1)],
                        idx_v.at[pl.ds(_CHUNKS_LO, 1)])

    bufs = (rows0, rows1)
    outs = (oc0, oc1)
    gsems = (gsem0, gsem1)
    ssems = (ssem0, ssem1)

    def start(c, b):
        pltpu.async_copy(theta_hbm.at[idx_v.at[c]], bufs[b], gsems[b])

    def reduce(b):
        rows = bufs[b]
        for bb in range(_B_PER_CHUNK):
            for g in range(EMBED_DIM // 32):
                los, his = [], []
                for k in range(NUM_NEIGHBORS):
                    e, o = plsc.unpack(
                        rows[bb * NUM_NEIGHBORS + k, pl.ds(g * 32, 32)],
                        format=plsc.PackFormat.INTERLEAVED)
                    los.append(e)
                    his.append(o)
                while len(los) > 1:
                    los = [los[i] + los[i + 1] for i in range(0, len(los), 2)]
                    his = [his[i] + his[i + 1] for i in range(0, len(his), 2)]
                outs[b][bb, pl.ds(g * 32, 16)] = los[0] * (1.0 / NUM_NEIGHBORS)
                outs[b][bb, pl.ds(g * 32 + 16, 16)] = (
                    his[0] * (1.0 / NUM_NEIGHBORS))

    for b in range(_NBUF):
        @pl.when(b < n_chunks)
        def _(b=b):
            start(b, b)

    def step(c, _):
        b = lax.rem(c, _NBUF)
        for bs in range(_NBUF):
            @pl.when(b == bs)
            def _(bs=bs):
                pltpu.make_async_copy(theta_hbm.at[idx_v.at[c]], bufs[bs],
                                      gsems[bs]).wait()

                @pl.when(c >= _NBUF)
                def _():
                    pltpu.make_async_copy(
                        outs[bs],
                        out_hbm.at[pl.ds(0, _B_PER_CHUNK)],
                        ssems[bs]).wait()

                reduce(bs)
                pltpu.async_copy(
                    outs[bs],
                    out_hbm.at[pl.ds((start_chunk + c) * _B_PER_CHUNK,
                                     _B_PER_CHUNK)],
                    ssems[bs])

                @pl.when(c + _NBUF < n_chunks)
                def _():
                    start(c + _NBUF, bs)
        return ()

    lax.fori_loop(0, n_chunks, step, (), unroll=False)
    for b in range(_NBUF):
        @pl.when(b < n_chunks)
        def _(b=b):
            pltpu.make_async_copy(outs[b],
                                  out_hbm.at[pl.ds(0, _B_PER_CHUNK)],
                                  ssems[b]).wait()


@jax.jit
def kernel(node_ids, neighbor_lists, theta):
    del node_ids
    theta_bf = (theta.astype(jnp.bfloat16)
                .reshape(NUM_NODES, 4, 2, 16)
                .transpose(0, 1, 3, 2)
                .reshape(NUM_NODES, EMBED_DIM))
    idx = neighbor_lists.reshape(_N_CHUNKS, _CHUNK_IDX)

    mesh = plsc.VectorSubcoreMesh(core_axis_name="c", subcore_axis_name="s")
    out = pl.kernel(
        _tec_body,
        out_type=jax.ShapeDtypeStruct((BATCH, EMBED_DIM), jnp.float32),
        mesh=mesh,
        compiler_params=pltpu.CompilerParams(needs_layout_passes=False,
                                             use_tc_tiling_on_sc=False),
        scratch_types=[
            pltpu.VMEM((_CHUNKS_LO + 1, _CHUNK_IDX), jnp.int32),
            pltpu.VMEM((_CHUNK_IDX, EMBED_DIM), jnp.bfloat16),
            pltpu.VMEM((_CHUNK_IDX, EMBED_DIM), jnp.bfloat16),
            pltpu.VMEM((_B_PER_CHUNK, EMBED_DIM), jnp.float32),
            pltpu.VMEM((_B_PER_CHUNK, EMBED_DIM), jnp.float32),
            pltpu.SemaphoreType.DMA,
            pltpu.SemaphoreType.DMA,
            pltpu.SemaphoreType.DMA,
            pltpu.SemaphoreType.DMA,
        ],
    )(theta_bf, idx)
    return out
